# baseline (speedup 1.0000x reference)
import math

import jax
import jax.numpy as jnp
from jax.experimental import pallas as pl

D = 768
DEPTH = 12
NH = 12
DK = 64
FF = 3072
E = 8
B = 4
S = 197
T = B * S
C = int(1.25 * T * 2 / E)
EC = E * C
NPATCH = 196
PDIM = 768
NCLS = 1000

f32 = jnp.float32


def _ln(x, s, b):
    m = jnp.mean(x, axis=-1, keepdims=True)
    v = jnp.mean((x - m) ** 2, axis=-1, keepdims=True)
    return (x - m) * jax.lax.rsqrt(v + 1e-6) * s + b


def _erf(x):
    a1, a2, a3, a4, a5 = (0.254829592, -0.284496736, 1.421413741,
                          -1.453152027, 1.061405429)
    p = 0.3275911
    ax = jnp.abs(x)
    t = 1.0 / (1.0 + p * ax)
    y = 1.0 - (((((a5 * t + a4) * t) + a3) * t + a2) * t + a1) * t * jnp.exp(-ax * ax)
    return jnp.sign(x) * y


def _gelu(x):
    return 0.5 * x * (1.0 + _erf(x * (1.0 / math.sqrt(2.0))))


def _patch_body(p_ref, w_ref, b_ref, pos_ref, o_ref):
    o_ref[...] = (jnp.dot(p_ref[...], w_ref[...], preferred_element_type=f32)
                  + b_ref[...] + pos_ref[...])


def _patch_embed(p, w, b, pos_tiled):
    return pl.pallas_call(
        _patch_body,
        out_shape=jax.ShapeDtypeStruct((B * NPATCH, D), f32),
    )(p, w, b.reshape(1, D), pos_tiled)


def _ln_mm_body(x_ref, s_ref, b_ref, w_ref, wb_ref, o_ref):
    xn = _ln(x_ref[...], s_ref[...], b_ref[...])
    o_ref[...] = jnp.dot(xn, w_ref[...], preferred_element_type=f32) + wb_ref[...]


def _ln_mm(x, s, b, w, wb):
    n = w.shape[1]
    return pl.pallas_call(
        _ln_mm_body,
        out_shape=jax.ShapeDtypeStruct((x.shape[0], n), f32),
    )(x, s.reshape(1, D), b.reshape(1, D), w, wb.reshape(1, n))


def _attn_body(q_ref, k_ref, v_ref, o_ref):
    q = q_ref[0]
    k = k_ref[0]
    v = v_ref[0]
    sc = jax.lax.dot_general(q, k, (((1,), (1,)), ((), ())),
                             preferred_element_type=f32) * (1.0 / math.sqrt(DK))
    sc = sc - jnp.max(sc, axis=-1, keepdims=True)
    ex = jnp.exp(sc)
    a = ex / jnp.sum(ex, axis=-1, keepdims=True)
    o_ref[0] = jnp.dot(a, v, preferred_element_type=f32)


def _attention(q, k, v):
    spec = pl.BlockSpec((1, S, DK), lambda i: (i, 0, 0))
    return pl.pallas_call(
        _attn_body,
        grid=(B * NH,),
        in_specs=[spec, spec, spec],
        out_specs=spec,
        out_shape=jax.ShapeDtypeStruct((B * NH, S, DK), f32),
    )(q, k, v)


def _res_mm_body(h_ref, x_ref, w_ref, b_ref, o_ref):
    o_ref[...] = (h_ref[...]
                  + jnp.dot(x_ref[...], w_ref[...], preferred_element_type=f32)
                  + b_ref[...])


def _res_mm(h, x, w, b):
    return pl.pallas_call(
        _res_mm_body,
        out_shape=jax.ShapeDtypeStruct((T, D), f32),
    )(h, x, w, b.reshape(1, D))


def _mlp_body(h_ref, s_ref, b_ref, w1_ref, b1_ref, w2_ref, b2_ref, o_ref):
    xn = _ln(h_ref[...], s_ref[...], b_ref[...])
    hh = _gelu(jnp.dot(xn, w1_ref[...], preferred_element_type=f32) + b1_ref[...])
    o_ref[...] = (h_ref[...]
                  + jnp.dot(hh, w2_ref[...], preferred_element_type=f32)
                  + b2_ref[...])


def _mlp(h, s, b, w1, b1, w2, b2):
    return pl.pallas_call(
        _mlp_body,
        out_shape=jax.ShapeDtypeStruct((T, D), f32),
    )(h, s.reshape(1, D), b.reshape(1, D), w1, b1.reshape(1, FF), w2,
      b2.reshape(1, D))


def _route_body(h_ref, s_ref, b_ref, gw_ref, xe_ref, comb_ref):
    xn = _ln(h_ref[...], s_ref[...], b_ref[...])
    logits = jnp.dot(xn, gw_ref[...], preferred_element_type=f32)
    logits = logits - jnp.max(logits, axis=-1, keepdims=True)
    pe = jnp.exp(logits)
    probs = pe / jnp.sum(pe, axis=-1, keepdims=True)

    ioE = jax.lax.broadcasted_iota(jnp.int32, (T, E), 1)
    mx1 = jnp.max(probs, axis=-1, keepdims=True)
    i1 = jnp.min(jnp.where(probs == mx1, ioE, E), axis=-1, keepdims=True)
    m1 = (ioE == i1).astype(f32)
    p2 = probs * (1.0 - m1)
    mx2 = jnp.max(p2, axis=-1, keepdims=True)
    i2 = jnp.min(jnp.where(p2 == mx2, ioE, E), axis=-1, keepdims=True)
    m2 = (ioE == i2).astype(f32)

    ri = jax.lax.broadcasted_iota(jnp.int32, (T, T), 0)
    rj = jax.lax.broadcasted_iota(jnp.int32, (T, T), 1)
    lower = (rj <= ri).astype(f32)
    pos1 = jnp.dot(lower, m1, preferred_element_type=f32) - 1.0
    cnt1 = jnp.sum(m1, axis=0, keepdims=True)
    pos2 = jnp.dot(lower, m2, preferred_element_type=f32) - 1.0 + cnt1

    keep1 = m1 * (pos1 < C).astype(f32)
    keep2 = m2 * (pos2 < C).astype(f32)
    g1 = jnp.sum(probs * keep1, axis=-1, keepdims=True)
    g2 = jnp.sum(probs * keep2, axis=-1, keepdims=True)
    denom = g1 + g2
    denom = jnp.where(denom > 0.0, denom, 1.0)
    g1 = g1 / denom
    g2 = g2 / denom

    k1 = jnp.sum(keep1, axis=-1, keepdims=True)
    k2 = jnp.sum(keep2, axis=-1, keepdims=True)
    slot1 = i1.astype(f32) * C + jnp.sum(pos1 * m1, axis=-1, keepdims=True)
    slot2 = i2.astype(f32) * C + jnp.sum(pos2 * m2, axis=-1, keepdims=True)

    ioEC = jax.lax.broadcasted_iota(jnp.int32, (T, EC), 1).astype(f32)
    d1 = (ioEC == slot1).astype(f32) * k1
    d2 = (ioEC == slot2).astype(f32) * k2
    disp = d1 + d2
    comb_ref[...] = g1 * d1 + g2 * d2
    xe_ref[...] = jax.lax.dot_general(disp, xn, (((0,), (0,)), ((), ())),
                                      preferred_element_type=f32)


def _route(h, s, b, gw):
    return pl.pallas_call(
        _route_body,
        out_shape=(jax.ShapeDtypeStruct((EC, D), f32),
                   jax.ShapeDtypeStruct((T, EC), f32)),
    )(h, s.reshape(1, D), b.reshape(1, D), gw)


def _expert_body(xe_ref, w1_ref, b1_ref, w2_ref, b2_ref, o_ref):
    hh = _gelu(jnp.dot(xe_ref[0], w1_ref[0], preferred_element_type=f32)
               + b1_ref[0])
    o_ref[0] = jnp.dot(hh, w2_ref[0], preferred_element_type=f32) + b2_ref[0]


def _experts(xe, ew1, eb1, ew2, eb2):
    out = pl.pallas_call(
        _expert_body,
        grid=(E,),
        in_specs=[
            pl.BlockSpec((1, C, D), lambda e: (e, 0, 0)),
            pl.BlockSpec((1, D, FF), lambda e: (e, 0, 0)),
            pl.BlockSpec((1, 1, FF), lambda e: (e, 0, 0)),
            pl.BlockSpec((1, FF, D), lambda e: (e, 0, 0)),
            pl.BlockSpec((1, 1, D), lambda e: (e, 0, 0)),
        ],
        out_specs=pl.BlockSpec((1, C, D), lambda e: (e, 0, 0)),
        out_shape=jax.ShapeDtypeStruct((E, C, D), f32),
    )(xe.reshape(E, C, D), ew1, eb1.reshape(E, 1, FF), ew2,
      eb2.reshape(E, 1, D))
    return out.reshape(EC, D)


def _combine_body(h_ref, comb_ref, eo_ref, o_ref):
    o_ref[...] = h_ref[...] + jnp.dot(comb_ref[...], eo_ref[...],
                                      preferred_element_type=f32)


def _combine(h, comb, eo):
    return pl.pallas_call(
        _combine_body,
        out_shape=jax.ShapeDtypeStruct((T, D), f32),
    )(h, comb, eo)


def _final_body(h_ref, s_ref, b_ref, cw_ref, cb_ref, o_ref):
    xn = _ln(h_ref[...], s_ref[...], b_ref[...])
    bi = jax.lax.broadcasted_iota(jnp.int32, (B, T), 0)
    tj = jax.lax.broadcasted_iota(jnp.int32, (B, T), 1)
    sel = ((tj >= bi * S) & (tj < (bi + 1) * S)).astype(f32) * (1.0 / S)
    pooled = jnp.dot(sel, xn, preferred_element_type=f32)
    o_ref[...] = jax.lax.dot_general(pooled, cw_ref[...],
                                     (((1,), (1,)), ((), ())),
                                     preferred_element_type=f32) + cb_ref[...]


def _final(h, s, b, cw, cb):
    return pl.pallas_call(
        _final_body,
        out_shape=jax.ShapeDtypeStruct((B, NCLS), f32),
    )(h, s.reshape(1, D), b.reshape(1, D), cw, cb.reshape(1, NCLS))


def kernel(x, params):
    gh = 224 // 16
    p = (x.reshape(B, 3, gh, 16, gh, 16)
          .transpose(0, 2, 4, 1, 3, 5)
          .reshape(B * NPATCH, PDIM))
    pos = params['pos']
    pos_tiled = jnp.broadcast_to(pos[:, 1:, :], (B, NPATCH, D)).reshape(B * NPATCH, D)
    hp = _patch_embed(p, params['patch_w'], params['patch_b'], pos_tiled)
    cls = jnp.broadcast_to(params['cls'] + pos[:, :1, :], (B, 1, D))
    h = jnp.concatenate([cls, hp.reshape(B, NPATCH, D)], axis=1).reshape(T, D)

    for l in range(DEPTH):
        lp = params['layers'][l]
        qkv = _ln_mm(h, lp['ln1_s'], lp['ln1_b'], lp['qkv_w'], lp['qkv_b'])
        qkv = (qkv.reshape(B, S, 3, NH, DK)
                  .transpose(2, 0, 3, 1, 4)
                  .reshape(3, B * NH, S, DK))
        o = _attention(qkv[0], qkv[1], qkv[2])
        o = (o.reshape(B, NH, S, DK).transpose(0, 2, 1, 3).reshape(T, NH * DK))
        h = _res_mm(h, o, lp['o_w'], lp['o_b'])
        if l % 2 == 0:
            h = _mlp(h, lp['ln2_s'], lp['ln2_b'], lp['w1'], lp['b1'],
                     lp['w2'], lp['b2'])
        else:
            xe, comb = _route(h, lp['ln2_s'], lp['ln2_b'], lp['gate_w'])
            eo = _experts(xe, lp['ew1'], lp['eb1'], lp['ew2'], lp['eb2'])
            h = _combine(h, comb, eo)

    return _final(h, params['lnf_s'], params['lnf_b'],
                  params['clf_w'], params['clf_b'])


# speedup vs baseline: 1.4729x; 1.4729x over previous
import math

import jax
import jax.numpy as jnp
from jax.experimental import pallas as pl

D = 768
DEPTH = 12
NH = 12
DK = 64
FF = 3072
E = 8
B = 4
S = 197
T = B * S
C = int(1.25 * T * 2 / E)
EC = E * C
NPATCH = 196
PDIM = 768
NCLS = 1000

f32 = jnp.float32


def _ln(x, s, b):
    m = jnp.mean(x, axis=-1, keepdims=True)
    v = jnp.mean((x - m) ** 2, axis=-1, keepdims=True)
    return (x - m) * jax.lax.rsqrt(v + 1e-6) * s + b


def _erf(x):
    a1, a2, a3, a4, a5 = (0.254829592, -0.284496736, 1.421413741,
                          -1.453152027, 1.061405429)
    p = 0.3275911
    ax = jnp.abs(x)
    t = 1.0 / (1.0 + p * ax)
    y = 1.0 - (((((a5 * t + a4) * t) + a3) * t + a2) * t + a1) * t * jnp.exp(-ax * ax)
    return jnp.sign(x) * y


def _gelu(x):
    return 0.5 * x * (1.0 + _erf(x * (1.0 / math.sqrt(2.0))))


def _patch_body(p_ref, w_ref, b_ref, pos_ref, o_ref):
    o_ref[...] = (jnp.dot(p_ref[...], w_ref[...], preferred_element_type=f32)
                  + b_ref[...] + pos_ref[...])


def _patch_embed(p, w, b, pos_tiled):
    return pl.pallas_call(
        _patch_body,
        out_shape=jax.ShapeDtypeStruct((B * NPATCH, D), f32),
    )(p, w, b.reshape(1, D), pos_tiled)


def _ln_mm_body(x_ref, s_ref, b_ref, w_ref, wb_ref, o_ref):
    xn = _ln(x_ref[...], s_ref[...], b_ref[...])
    o_ref[...] = jnp.dot(xn, w_ref[...], preferred_element_type=f32) + wb_ref[...]


def _ln_mm(x, s, b, w, wb):
    n = w.shape[1]
    return pl.pallas_call(
        _ln_mm_body,
        out_shape=jax.ShapeDtypeStruct((x.shape[0], n), f32),
    )(x, s.reshape(1, D), b.reshape(1, D), w, wb.reshape(1, n))


def _attn_body(q_ref, k_ref, v_ref, o_ref):
    scale = 1.0 / math.sqrt(DK)
    for i in range(2):
        q = q_ref[0][:, i * DK:(i + 1) * DK]
        k = k_ref[0][:, i * DK:(i + 1) * DK]
        v = v_ref[0][:, i * DK:(i + 1) * DK]
        sc = jax.lax.dot_general(q, k, (((1,), (1,)), ((), ())),
                                 preferred_element_type=f32) * scale
        sc = sc - jnp.max(sc, axis=-1, keepdims=True)
        ex = jnp.exp(sc)
        a = ex / jnp.sum(ex, axis=-1, keepdims=True)
        o_ref[0, :, i * DK:(i + 1) * DK] = jnp.dot(a, v,
                                                   preferred_element_type=f32)


def _attention(qkv3):
    hp = NH // 2
    blk = (1, S, 2 * DK)
    return pl.pallas_call(
        _attn_body,
        grid=(B, hp),
        in_specs=[
            pl.BlockSpec(blk, lambda b, h: (b, 0, h)),
            pl.BlockSpec(blk, lambda b, h: (b, 0, hp + h)),
            pl.BlockSpec(blk, lambda b, h: (b, 0, 2 * hp + h)),
        ],
        out_specs=pl.BlockSpec(blk, lambda b, h: (b, 0, h)),
        out_shape=jax.ShapeDtypeStruct((B, S, D), f32),
    )(qkv3, qkv3, qkv3)


def _dense_tail_body(h_ref, att_ref, ow_ref, ob_ref, s_ref, b_ref,
                     w1_ref, b1_ref, w2_ref, b2_ref, o_ref):
    h2 = (h_ref[...]
          + jnp.dot(att_ref[...], ow_ref[...], preferred_element_type=f32)
          + ob_ref[...])
    xn = _ln(h2, s_ref[...], b_ref[...])
    hh = _gelu(jnp.dot(xn, w1_ref[...], preferred_element_type=f32) + b1_ref[...])
    o_ref[...] = (h2 + jnp.dot(hh, w2_ref[...], preferred_element_type=f32)
                  + b2_ref[...])


def _dense_tail(h, att, ow, ob, s, b, w1, b1, w2, b2):
    return pl.pallas_call(
        _dense_tail_body,
        out_shape=jax.ShapeDtypeStruct((T, D), f32),
    )(h, att, ow, ob.reshape(1, D), s.reshape(1, D), b.reshape(1, D),
      w1, b1.reshape(1, FF), w2, b2.reshape(1, D))


def _moe_tail_body(h_ref, att_ref, ow_ref, ob_ref, s_ref, b_ref, gw_ref,
                   h2_ref, xe_ref, comb_ref):
    h2 = (h_ref[...]
          + jnp.dot(att_ref[...], ow_ref[...], preferred_element_type=f32)
          + ob_ref[...])
    h2_ref[...] = h2
    xn = _ln(h2, s_ref[...], b_ref[...])
    logits = jnp.dot(xn, gw_ref[...], preferred_element_type=f32)
    logits = logits - jnp.max(logits, axis=-1, keepdims=True)
    pe = jnp.exp(logits)
    probs = pe / jnp.sum(pe, axis=-1, keepdims=True)

    ioE = jax.lax.broadcasted_iota(jnp.int32, (T, E), 1)
    mx1 = jnp.max(probs, axis=-1, keepdims=True)
    i1 = jnp.min(jnp.where(probs == mx1, ioE, E), axis=-1, keepdims=True)
    m1 = (ioE == i1).astype(f32)
    p2 = probs * (1.0 - m1)
    mx2 = jnp.max(p2, axis=-1, keepdims=True)
    i2 = jnp.min(jnp.where(p2 == mx2, ioE, E), axis=-1, keepdims=True)
    m2 = (ioE == i2).astype(f32)

    ri = jax.lax.broadcasted_iota(jnp.int32, (T, T), 0)
    rj = jax.lax.broadcasted_iota(jnp.int32, (T, T), 1)
    lower = (rj <= ri).astype(f32)
    pos1 = jnp.dot(lower, m1, preferred_element_type=f32) - 1.0
    cnt1 = jnp.sum(m1, axis=0, keepdims=True)
    pos2 = jnp.dot(lower, m2, preferred_element_type=f32) - 1.0 + cnt1

    keep1 = m1 * (pos1 < C).astype(f32)
    keep2 = m2 * (pos2 < C).astype(f32)
    g1 = jnp.sum(probs * keep1, axis=-1, keepdims=True)
    g2 = jnp.sum(probs * keep2, axis=-1, keepdims=True)
    denom = g1 + g2
    denom = jnp.where(denom > 0.0, denom, 1.0)
    g1 = g1 / denom
    g2 = g2 / denom

    k1 = jnp.sum(keep1, axis=-1, keepdims=True)
    k2 = jnp.sum(keep2, axis=-1, keepdims=True)
    slot1 = i1.astype(f32) * C + jnp.sum(pos1 * m1, axis=-1, keepdims=True)
    slot2 = i2.astype(f32) * C + jnp.sum(pos2 * m2, axis=-1, keepdims=True)

    ioEC = jax.lax.broadcasted_iota(jnp.int32, (T, EC), 1).astype(f32)
    d1 = (ioEC == slot1).astype(f32) * k1
    d2 = (ioEC == slot2).astype(f32) * k2
    disp = d1 + d2
    comb_ref[...] = g1 * d1 + g2 * d2
    xe_ref[...] = jax.lax.dot_general(disp, xn, (((0,), (0,)), ((), ())),
                                      preferred_element_type=f32)


def _moe_tail(h, att, ow, ob, s, b, gw):
    return pl.pallas_call(
        _moe_tail_body,
        out_shape=(jax.ShapeDtypeStruct((T, D), f32),
                   jax.ShapeDtypeStruct((EC, D), f32),
                   jax.ShapeDtypeStruct((T, EC), f32)),
    )(h, att, ow, ob.reshape(1, D), s.reshape(1, D), b.reshape(1, D), gw)


def _expert_body(xe_ref, w1_ref, b1_ref, w2_ref, b2_ref, o_ref):
    hh = _gelu(jnp.dot(xe_ref[0], w1_ref[0], preferred_element_type=f32)
               + b1_ref[0])
    o_ref[0] = jnp.dot(hh, w2_ref[0], preferred_element_type=f32) + b2_ref[0]


def _experts(xe, ew1, eb1, ew2, eb2):
    out = pl.pallas_call(
        _expert_body,
        grid=(E,),
        in_specs=[
            pl.BlockSpec((1, C, D), lambda e: (e, 0, 0)),
            pl.BlockSpec((1, D, FF), lambda e: (e, 0, 0)),
            pl.BlockSpec((1, 1, FF), lambda e: (e, 0, 0)),
            pl.BlockSpec((1, FF, D), lambda e: (e, 0, 0)),
            pl.BlockSpec((1, 1, D), lambda e: (e, 0, 0)),
        ],
        out_specs=pl.BlockSpec((1, C, D), lambda e: (e, 0, 0)),
        out_shape=jax.ShapeDtypeStruct((E, C, D), f32),
    )(xe.reshape(E, C, D), ew1, eb1.reshape(E, 1, FF), ew2,
      eb2.reshape(E, 1, D))
    return out.reshape(EC, D)


def _combine_body(h_ref, comb_ref, eo_ref, o_ref):
    o_ref[...] = h_ref[...] + jnp.dot(comb_ref[...], eo_ref[...],
                                      preferred_element_type=f32)


def _combine(h, comb, eo):
    return pl.pallas_call(
        _combine_body,
        out_shape=jax.ShapeDtypeStruct((T, D), f32),
    )(h, comb, eo)


def _final_body(h_ref, s_ref, b_ref, cw_ref, cb_ref, o_ref):
    xn = _ln(h_ref[...], s_ref[...], b_ref[...])
    bi = jax.lax.broadcasted_iota(jnp.int32, (B, T), 0)
    tj = jax.lax.broadcasted_iota(jnp.int32, (B, T), 1)
    sel = ((tj >= bi * S) & (tj < (bi + 1) * S)).astype(f32) * (1.0 / S)
    pooled = jnp.dot(sel, xn, preferred_element_type=f32)
    o_ref[...] = jax.lax.dot_general(pooled, cw_ref[...],
                                     (((1,), (1,)), ((), ())),
                                     preferred_element_type=f32) + cb_ref[...]


def _final(h, s, b, cw, cb):
    return pl.pallas_call(
        _final_body,
        out_shape=jax.ShapeDtypeStruct((B, NCLS), f32),
    )(h, s.reshape(1, D), b.reshape(1, D), cw, cb.reshape(1, NCLS))


def kernel(x, params):
    gh = 224 // 16
    p = (x.reshape(B, 3, gh, 16, gh, 16)
          .transpose(0, 2, 4, 1, 3, 5)
          .reshape(B * NPATCH, PDIM))
    pos = params['pos']
    pos_tiled = jnp.broadcast_to(pos[:, 1:, :], (B, NPATCH, D)).reshape(B * NPATCH, D)
    hp = _patch_embed(p, params['patch_w'], params['patch_b'], pos_tiled)
    cls = jnp.broadcast_to(params['cls'] + pos[:, :1, :], (B, 1, D))
    h = jnp.concatenate([cls, hp.reshape(B, NPATCH, D)], axis=1).reshape(T, D)

    for l in range(DEPTH):
        lp = params['layers'][l]
        qkv = _ln_mm(h, lp['ln1_s'], lp['ln1_b'], lp['qkv_w'], lp['qkv_b'])
        att = _attention(qkv.reshape(B, S, 3 * D)).reshape(T, D)
        if l % 2 == 0:
            h = _dense_tail(h, att, lp['o_w'], lp['o_b'], lp['ln2_s'],
                            lp['ln2_b'], lp['w1'], lp['b1'], lp['w2'], lp['b2'])
        else:
            h2, xe, comb = _moe_tail(h, att, lp['o_w'], lp['o_b'],
                                     lp['ln2_s'], lp['ln2_b'], lp['gate_w'])
            eo = _experts(xe, lp['ew1'], lp['eb1'], lp['ew2'], lp['eb2'])
            h = _combine(h2, comb, eo)

    return _final(h, params['lnf_s'], params['lnf_b'],
                  params['clf_w'], params['clf_b'])
